# baseline (speedup 1.0000x reference)
import functools

import jax
import jax.numpy as jnp
from jax import lax
from jax.experimental import pallas as pl
from jax.experimental.pallas import tpu as pltpu
from jax.experimental.pallas import tpu_sc as plsc

B, N, C_PTS = 8, 4096, 128
S, K = 512, 32
R2 = 0.2 * 0.2
M = B * S * K
EPS = 1e-5

NC, NS = 2, 16
NW = NC * NS
ROWS_PER_W = (B * S) // NW
CH = 4


def _fps_body(p24_ref, nout_ref):
    p24 = p24_ref[...]
    x = p24[0:8]
    y = p24[8:16]
    z = p24[16:24]
    lane = lax.broadcasted_iota(jnp.int32, (8, N), 1)
    lane24 = lax.broadcasted_iota(jnp.int32, (24, N), 1)

    def body(i, carry):
        dist, f = carry
        f24 = jnp.concatenate([f, f, f], axis=0)
        c24 = jnp.sum(jnp.where(lane24 == f24, p24, 0.0),
                      axis=-1, keepdims=True)
        nout_ref[pl.ds(i, 1), :] = jnp.transpose(c24)
        cx = c24[0:8]
        cy = c24[8:16]
        cz = c24[16:24]
        d = (x - cx) ** 2 + (y - cy) ** 2 + (z - cz) ** 2
        dist = jnp.minimum(dist, d)
        m = jnp.max(dist, axis=-1, keepdims=True)
        f_next = jnp.min(jnp.where(dist == m, lane, N),
                         axis=-1, keepdims=True)
        return (dist, f_next)

    init = (jnp.full((8, N), 1e10, jnp.float32),
            jnp.zeros((8, 1), jnp.int32))
    lax.fori_loop(0, S, body, init)


def _run_fps(p24):
    return pl.pallas_call(
        _fps_body,
        grid=(1,),
        in_specs=[pl.BlockSpec((24, N), lambda i: (0, 0))],
        out_specs=pl.BlockSpec((S, 24), lambda i: (0, 0)),
        out_shape=jax.ShapeDtypeStruct((S, 24), jnp.float32),
    )(p24)


def _z_body(xyz_ref, pts_ref, w1x_ref, w1p_ref, b1_ref, z_ref):
    xz = xyz_ref[0]
    pts = pts_ref[0]
    zz = lax.dot_general(pts, w1p_ref[...], (((1,), (1,)), ((), ())),
                         preferred_element_type=jnp.float32)
    zz = zz + lax.dot_general(xz, w1x_ref[...], (((1,), (1,)), ((), ())),
                              preferred_element_type=jnp.float32)
    z_ref[0] = zz + b1_ref[...]


def _run_z(xyz, points, w1x, w1p, b1):
    return pl.pallas_call(
        _z_body,
        grid=(B, N // 512),
        in_specs=[
            pl.BlockSpec((1, 512, 3), lambda b, j: (b, j, 0)),
            pl.BlockSpec((1, 512, C_PTS), lambda b, j: (b, j, 0)),
            pl.BlockSpec((128, 3), lambda b, j: (0, 0)),
            pl.BlockSpec((128, 128), lambda b, j: (0, 0)),
            pl.BlockSpec((1, 128), lambda b, j: (0, 0)),
        ],
        out_specs=pl.BlockSpec((1, 512, 128), lambda b, j: (b, j, 0)),
        out_shape=jax.ShapeDtypeStruct((B, N, 128), jnp.float32),
    )(xyz, points, w1x, w1p, b1)


SEGW = N // 2


def _dist_body(nxyz_ref, xyz_ref, xyzt_ref, w1x_ref, dist_ref, q_ref):
    nx = nxyz_ref[0]
    xz = xyz_ref[0]
    mm = lax.dot_general(nx.astype(jnp.bfloat16), xz.astype(jnp.bfloat16),
                         (((1,), (1,)), ((), ())),
                         preferred_element_type=jnp.float32)
    q2 = (nx[:, 0:1] * nx[:, 0:1] + nx[:, 1:2] * nx[:, 1:2]) \
        + nx[:, 2:3] * nx[:, 2:3]
    xp = xyzt_ref[0, 0:1, :]
    yp = xyzt_ref[0, 1:2, :]
    zp = xyzt_ref[0, 2:3, :]
    x2row = (xp * xp + yp * yp) + zp * zp
    d = (-2.0) * mm
    d = d + q2
    d = d + x2row
    dist_ref[0, 0] = d

    @pl.when(pl.program_id(1) == 0)
    def _():
        q_ref[0] = lax.dot_general(nx, w1x_ref[...], (((1,), (1,)), ((), ())),
                                   preferred_element_type=jnp.float32)


def _run_dist(new_xyz, xyz, xyzt3, w1x):
    return pl.pallas_call(
        _dist_body,
        grid=(B, 2),
        in_specs=[
            pl.BlockSpec((1, S, 3), lambda b, h: (b, 0, 0)),
            pl.BlockSpec((1, SEGW, 3), lambda b, h: (b, h, 0)),
            pl.BlockSpec((1, 3, SEGW), lambda b, h: (b, 0, h)),
            pl.BlockSpec((128, 3), lambda b, h: (0, 0)),
        ],
        out_specs=[
            pl.BlockSpec((1, 1, S, SEGW), lambda b, h: (h, b, 0, 0)),
            pl.BlockSpec((1, S, 128), lambda b, h: (b, 0, 0)),
        ],
        out_shape=[
            jax.ShapeDtypeStruct((2, B, S, SEGW), jnp.float32),
            jax.ShapeDtypeStruct((B, S, 128), jnp.float32),
        ],
    )(new_xyz, xyz, xyzt3, w1x)


def _sc_body(dist_hbm, z_hbm, out_hbm, drow, ib0, ib1, ib2, ib3,
             is0, is1, is2, is3, gbuf, sem):
    wid = lax.axis_index("s") * NC + lax.axis_index("c")
    base_s = wid * ROWS_PER_W
    zbase = (base_s // S) * N
    iota16 = lax.iota(jnp.int32, 16)
    ibuf = [ib0, ib1, ib2, ib3]
    isel = [is0, is1, is2, is3]
    big = jnp.full((16,), 1 << 30, jnp.int32)

    def make_superchunk(ci, h):
        pbase = h * SEGW + zbase

        def scan_body(j, c2):
            cnt, first = c2
            d = drow[ci, pl.ds(j * 16, 16)]
            m = jnp.logical_and(d <= R2, cnt < K)
            key = jnp.where(m, iota16 + (j * 16 + pbase), big)
            sk, _ = plsc.sort_key_val(key, key)
            pc = plsc.all_reduce_population_count(m)
            ffs = plsc.all_reduce_ffs(m)
            cand = jnp.where(pc > 0, (j * 16 + pbase) + ffs, big)
            plsc.store_scatter(ibuf[ci], [cnt + iota16], sk)
            return (cnt + jnp.where(cnt < K, pc, 0),
                    jnp.minimum(first, cand))

        def superchunk(sj, c):
            cnt_s = jnp.max(c[0], axis=0)

            def live(c2):
                for jj in range(16):
                    c2 = scan_body(sj * 16 + jj, c2)
                return c2

            return lax.cond(cnt_s < K, live, lambda c2: c2, c)

        return superchunk

    def scan_rows(h, flat):
        out = []
        for ci in range(CH):
            c = (flat[2 * ci], flat[2 * ci + 1])
            c = lax.fori_loop(0, SEGW // 256, make_superchunk(ci, h), c)
            out.extend(c)
        return tuple(out)

    def chunk(cidx, _):
        srow0 = base_s + cidx * CH
        pltpu.sync_copy(dist_hbm.at[0, pl.ds(srow0, CH)], drow)
        flat = (jnp.zeros((16,), jnp.int32), big) * CH
        flat = scan_rows(0, flat)
        alive = jnp.max(flat[0], axis=0) < K
        for ci in range(1, CH):
            alive = jnp.logical_or(alive, jnp.max(flat[2 * ci], axis=0) < K)

        def second(fl):
            pltpu.sync_copy(dist_hbm.at[1, pl.ds(srow0, CH)], drow)
            return scan_rows(1, fl)

        flat = lax.cond(alive, second, lambda fl: fl, flat)

        copies = []
        for ci in range(CH):
            cntv, firstv = flat[2 * ci], flat[2 * ci + 1]
            for hh in range(2):
                v = ibuf[ci][pl.ds(hh * 16, 16)]
                pos = iota16 + (hh * 16)
                isel[ci][pl.ds(hh * 16, 16)] = jnp.where(pos < cntv, v, firstv)
            copies.append(pltpu.async_copy(
                z_hbm.at[isel[ci]], gbuf.at[pl.ds(ci * K, K)], sem))
        for cp in copies:
            cp.wait()
        pltpu.sync_copy(gbuf, out_hbm.at[pl.ds(srow0 * K, CH * K)])
        return _

    lax.fori_loop(0, ROWS_PER_W // CH, chunk, 0)


def _sc_gather(dist3d, z2d):
    mesh = plsc.VectorSubcoreMesh(core_axis_name="c", subcore_axis_name="s")
    fn = functools.partial(
        pl.kernel,
        mesh=mesh,
        compiler_params=pltpu.CompilerParams(needs_layout_passes=False),
        out_type=jax.ShapeDtypeStruct((M, 128), jnp.float32),
        scratch_types=[
            pltpu.VMEM((CH, SEGW), jnp.float32),
            pltpu.VMEM((48,), jnp.int32),
            pltpu.VMEM((48,), jnp.int32),
            pltpu.VMEM((48,), jnp.int32),
            pltpu.VMEM((48,), jnp.int32),
            pltpu.VMEM((K,), jnp.int32),
            pltpu.VMEM((K,), jnp.int32),
            pltpu.VMEM((K,), jnp.int32),
            pltpu.VMEM((K,), jnp.int32),
            pltpu.VMEM((CH * K, 128), jnp.float32),
            pltpu.SemaphoreType.DMA,
        ],
    )(_sc_body)
    return fn(dist3d, z2d)


def _p1_body(g_ref, q_ref, sums_ref):
    i = pl.program_id(0)
    x1 = g_ref[...].reshape(16, K, 128) - q_ref[...].reshape(16, 1, 128)
    x1 = x1.reshape(512, 128)
    s = jnp.sum(x1, axis=0, keepdims=True)
    sq = jnp.sum(x1 * x1, axis=0, keepdims=True)

    @pl.when(i == 0)
    def _():
        sums_ref[...] = jnp.zeros_like(sums_ref)

    sums_ref[0:1, :] += s
    sums_ref[1:2, :] += sq


def _run_p1(g, q):
    return pl.pallas_call(
        _p1_body,
        grid=(M // 512,),
        in_specs=[
            pl.BlockSpec((512, 128), lambda i: (i, 0)),
            pl.BlockSpec((16, 128), lambda i: (i, 0)),
        ],
        out_specs=pl.BlockSpec((8, 128), lambda i: (0, 0)),
        out_shape=jax.ShapeDtypeStruct((8, 128), jnp.float32),
    )(g, q)


def _bn_scale_shift(sums_ref, g_ref, be_ref, m_count):
    mean = sums_ref[0:1, :] * (1.0 / m_count)
    var = sums_ref[1:2, :] * (1.0 / m_count) - mean * mean
    rstd = lax.rsqrt(var + EPS)
    scale = g_ref[...] * rstd
    shift = be_ref[...] - mean * scale
    return scale, shift


def _p2_body(g_ref, q_ref, sums_ref, g1_ref, be1_ref, w2_ref, b2_ref,
             x2_ref, sums2_ref):
    i = pl.program_id(0)
    scale, shift = _bn_scale_shift(sums_ref, g1_ref, be1_ref, float(M))
    x1 = g_ref[...].reshape(16, K, 128) - q_ref[...].reshape(16, 1, 128)
    x1 = x1.reshape(512, 128)
    x1 = jnp.maximum(x1 * scale + shift, 0.0)
    y = lax.dot_general(x1.astype(jnp.bfloat16),
                        w2_ref[...].astype(jnp.bfloat16),
                        (((1,), (1,)), ((), ())),
                        preferred_element_type=jnp.float32) + b2_ref[...]
    x2_ref[...] = y.astype(jnp.bfloat16)
    s = jnp.sum(y, axis=0, keepdims=True)
    sq = jnp.sum(y * y, axis=0, keepdims=True)

    @pl.when(i == 0)
    def _():
        sums2_ref[...] = jnp.zeros_like(sums2_ref)

    sums2_ref[0:1, :] += s
    sums2_ref[1:2, :] += sq


def _run_p2(g, q, sums1, g1, be1, w2, b2):
    return pl.pallas_call(
        _p2_body,
        grid=(M // 512,),
        in_specs=[
            pl.BlockSpec((512, 128), lambda i: (i, 0)),
            pl.BlockSpec((16, 128), lambda i: (i, 0)),
            pl.BlockSpec((8, 128), lambda i: (0, 0)),
            pl.BlockSpec((1, 128), lambda i: (0, 0)),
            pl.BlockSpec((1, 128), lambda i: (0, 0)),
            pl.BlockSpec((128, 128), lambda i: (0, 0)),
            pl.BlockSpec((1, 128), lambda i: (0, 0)),
        ],
        out_specs=[
            pl.BlockSpec((512, 128), lambda i: (i, 0)),
            pl.BlockSpec((8, 128), lambda i: (0, 0)),
        ],
        out_shape=[
            jax.ShapeDtypeStruct((M, 128), jnp.bfloat16),
            jax.ShapeDtypeStruct((8, 128), jnp.float32),
        ],
    )(g, q, sums1, g1, be1, w2, b2)


def _p3_body(x2_ref, sums_ref, g2_ref, be2_ref, w3_ref, b3_ref,
             x3_ref, sums3_ref):
    i = pl.program_id(0)
    scale, shift = _bn_scale_shift(sums_ref, g2_ref, be2_ref, float(M))
    x2 = jnp.maximum(x2_ref[...].astype(jnp.float32) * scale + shift, 0.0)
    y = lax.dot_general(x2.astype(jnp.bfloat16),
                        w3_ref[...].astype(jnp.bfloat16),
                        (((1,), (1,)), ((), ())),
                        preferred_element_type=jnp.float32) + b3_ref[...]
    x3_ref[...] = y.astype(jnp.bfloat16)
    s = jnp.sum(y, axis=0, keepdims=True)
    sq = jnp.sum(y * y, axis=0, keepdims=True)

    @pl.when(i == 0)
    def _():
        sums3_ref[...] = jnp.zeros_like(sums3_ref)

    sums3_ref[0:1, :] += s
    sums3_ref[1:2, :] += sq


def _run_p3(x2, sums2, g2, be2, w3, b3):
    return pl.pallas_call(
        _p3_body,
        grid=(M // 512,),
        in_specs=[
            pl.BlockSpec((512, 128), lambda i: (i, 0)),
            pl.BlockSpec((8, 128), lambda i: (0, 0)),
            pl.BlockSpec((1, 128), lambda i: (0, 0)),
            pl.BlockSpec((1, 128), lambda i: (0, 0)),
            pl.BlockSpec((256, 128), lambda i: (0, 0)),
            pl.BlockSpec((1, 256), lambda i: (0, 0)),
        ],
        out_specs=[
            pl.BlockSpec((512, 256), lambda i: (i, 0)),
            pl.BlockSpec((8, 256), lambda i: (0, 0)),
        ],
        out_shape=[
            jax.ShapeDtypeStruct((M, 256), jnp.bfloat16),
            jax.ShapeDtypeStruct((8, 256), jnp.float32),
        ],
    )(x2, sums2, g2, be2, w3, b3)


def _p4_body(x3_ref, sums_ref, g3_ref, be3_ref, out_ref):
    scale, shift = _bn_scale_shift(sums_ref, g3_ref, be3_ref, float(M))
    x3 = jnp.maximum(x3_ref[...].astype(jnp.float32) * scale + shift, 0.0)
    out_ref[...] = jnp.max(x3.reshape(16, K, 256), axis=1)


def _run_p4(x3, sums3, g3, be3):
    return pl.pallas_call(
        _p4_body,
        grid=(M // 512,),
        in_specs=[
            pl.BlockSpec((512, 256), lambda i: (i, 0)),
            pl.BlockSpec((8, 256), lambda i: (0, 0)),
            pl.BlockSpec((1, 256), lambda i: (0, 0)),
            pl.BlockSpec((1, 256), lambda i: (0, 0)),
        ],
        out_specs=pl.BlockSpec((16, 256), lambda i: (i, 0)),
        out_shape=jax.ShapeDtypeStruct((B * S, 256), jnp.float32),
    )(x3, sums3, g3, be3)


def kernel(xyz, points, W1, b1, g1, be1, W2, b2, g2, be2, W3, b3, g3, be3):
    w1x = W1[:, :3]
    w1p = W1[:, 3:]
    xyzt3 = jnp.transpose(xyz, (0, 2, 1))
    p24 = jnp.transpose(xyzt3, (1, 0, 2)).reshape(24, N)

    nout = _run_fps(p24)
    new_xyz = jnp.transpose(nout.reshape(S, 3, B), (2, 0, 1))
    z = _run_z(xyz, points, w1x, w1p, b1.reshape(1, 128))
    dist, q = _run_dist(new_xyz, xyz, xyzt3, w1x)

    g = _sc_gather(dist.reshape(2, B * S, SEGW), z.reshape(B * N, 128))

    q2d = q.reshape(B * S, 128)
    sums1 = _run_p1(g, q2d)
    x2, sums2 = _run_p2(g, q2d, sums1, g1.reshape(1, 128), be1.reshape(1, 128),
                        W2, b2.reshape(1, 128))
    x3, sums3 = _run_p3(x2, sums2, g2.reshape(1, 128), be2.reshape(1, 128),
                        W3, b3.reshape(1, 256))
    out = _run_p4(x3, sums3, g3.reshape(1, 256), be3.reshape(1, 256))
    return (new_xyz, out.reshape(B, S, 256))


# speedup vs baseline: 1.0271x; 1.0271x over previous
import functools

import jax
import jax.numpy as jnp
from jax import lax
from jax.experimental import pallas as pl
from jax.experimental.pallas import tpu as pltpu
from jax.experimental.pallas import tpu_sc as plsc

B, N, C_PTS = 8, 4096, 128
S, K = 512, 32
R2 = 0.2 * 0.2
M = B * S * K
EPS = 1e-5

NC, NS = 2, 16
NW = NC * NS
ROWS_PER_W = (B * S) // NW
CH = 4


def _fps_body(p24_ref, nout_ref):
    p24 = p24_ref[...]
    x = p24[0:8]
    y = p24[8:16]
    z = p24[16:24]
    lane = lax.broadcasted_iota(jnp.int32, (8, N), 1)
    lane24 = lax.broadcasted_iota(jnp.int32, (24, N), 1)

    def body(i, carry):
        dist, f = carry
        f24 = jnp.concatenate([f, f, f], axis=0)
        c24 = jnp.sum(jnp.where(lane24 == f24, p24, 0.0),
                      axis=-1, keepdims=True)
        nout_ref[pl.ds(i, 1), :] = jnp.transpose(c24)
        cx = c24[0:8]
        cy = c24[8:16]
        cz = c24[16:24]
        d = (x - cx) ** 2 + (y - cy) ** 2 + (z - cz) ** 2
        dist = jnp.minimum(dist, d)
        m = jnp.max(dist, axis=-1, keepdims=True)
        f_next = jnp.min(jnp.where(dist == m, lane, N),
                         axis=-1, keepdims=True)
        return (dist, f_next)

    init = (jnp.full((8, N), 1e10, jnp.float32),
            jnp.zeros((8, 1), jnp.int32))
    lax.fori_loop(0, S, body, init)


def _run_fps(p24):
    return pl.pallas_call(
        _fps_body,
        grid=(1,),
        in_specs=[pl.BlockSpec((24, N), lambda i: (0, 0))],
        out_specs=pl.BlockSpec((S, 24), lambda i: (0, 0)),
        out_shape=jax.ShapeDtypeStruct((S, 24), jnp.float32),
    )(p24)


def _z_body(xyz_ref, pts_ref, w1x_ref, w1p_ref, b1_ref, z_ref):
    xz = xyz_ref[0]
    pts = pts_ref[0]
    zz = lax.dot_general(pts, w1p_ref[...], (((1,), (1,)), ((), ())),
                         preferred_element_type=jnp.float32)
    zz = zz + lax.dot_general(xz, w1x_ref[...], (((1,), (1,)), ((), ())),
                              preferred_element_type=jnp.float32)
    z_ref[0] = zz + b1_ref[...]


def _run_z(xyz, points, w1x, w1p, b1):
    return pl.pallas_call(
        _z_body,
        grid=(B, N // 512),
        in_specs=[
            pl.BlockSpec((1, 512, 3), lambda b, j: (b, j, 0)),
            pl.BlockSpec((1, 512, C_PTS), lambda b, j: (b, j, 0)),
            pl.BlockSpec((128, 3), lambda b, j: (0, 0)),
            pl.BlockSpec((128, 128), lambda b, j: (0, 0)),
            pl.BlockSpec((1, 128), lambda b, j: (0, 0)),
        ],
        out_specs=pl.BlockSpec((1, 512, 128), lambda b, j: (b, j, 0)),
        out_shape=jax.ShapeDtypeStruct((B, N, 128), jnp.float32),
    )(xyz, points, w1x, w1p, b1)


def _dist_body(nxyz_ref, xyz_ref, xyzt_ref, w1x_ref, dist_ref, q_ref):
    nx = nxyz_ref[0]
    xz = xyz_ref[0]
    mm = lax.dot_general(nx.astype(jnp.bfloat16), xz.astype(jnp.bfloat16),
                         (((1,), (1,)), ((), ())),
                         preferred_element_type=jnp.float32)
    q2 = (nx[:, 0:1] * nx[:, 0:1] + nx[:, 1:2] * nx[:, 1:2]) \
        + nx[:, 2:3] * nx[:, 2:3]
    xp = xyzt_ref[0, 0:1, :]
    yp = xyzt_ref[0, 1:2, :]
    zp = xyzt_ref[0, 2:3, :]
    x2row = (xp * xp + yp * yp) + zp * zp
    d = (-2.0) * mm
    d = d + q2
    d = d + x2row
    dist_ref[0] = d
    q_ref[0] = lax.dot_general(nx, w1x_ref[...], (((1,), (1,)), ((), ())),
                               preferred_element_type=jnp.float32)


def _run_dist(new_xyz, xyz, xyzt3, w1x):
    return pl.pallas_call(
        _dist_body,
        grid=(B,),
        in_specs=[
            pl.BlockSpec((1, S, 3), lambda b: (b, 0, 0)),
            pl.BlockSpec((1, N, 3), lambda b: (b, 0, 0)),
            pl.BlockSpec((1, 3, N), lambda b: (b, 0, 0)),
            pl.BlockSpec((128, 3), lambda b: (0, 0)),
        ],
        out_specs=[
            pl.BlockSpec((1, S, N), lambda b: (b, 0, 0)),
            pl.BlockSpec((1, S, 128), lambda b: (b, 0, 0)),
        ],
        out_shape=[
            jax.ShapeDtypeStruct((B, S, N), jnp.float32),
            jax.ShapeDtypeStruct((B, S, 128), jnp.float32),
        ],
    )(new_xyz, xyz, xyzt3, w1x)


def _sc_body(dist_hbm, z_hbm, out_hbm, drow, idxbuf, is0, is1, is2, is3,
             gbuf, sem):
    wid = lax.axis_index("s") * NC + lax.axis_index("c")
    base_s = wid * ROWS_PER_W
    zbase = (base_s // S) * N
    iota16 = lax.iota(jnp.int32, 16)
    isel = [is0, is1, is2, is3]

    def chunk(cidx, _):
        srow0 = base_s + cidx * CH
        pltpu.sync_copy(dist_hbm.at[pl.ds(srow0, CH)], drow)
        copies = []
        for ci in range(CH):
            big = jnp.full((16,), 1 << 30, jnp.int32)

            def scan_body(j, carry):
                cnt, first = carry
                d = drow[ci, pl.ds(j * 16, 16)]
                m = jnp.logical_and(d <= R2, cnt < K)
                iv = iota16 + (j * 16 + zbase)
                key = jnp.where(m, iv, big)
                sk, _ = plsc.sort_key_val(key, key)
                pc = plsc.all_reduce_population_count(m)
                ffs = plsc.all_reduce_ffs(m)
                cand = jnp.where(pc > 0, (j * 16 + zbase) + ffs, big)
                plsc.store_scatter(idxbuf, [cnt + iota16], sk)
                return (cnt + jnp.where(cnt < K, pc, 0),
                        jnp.minimum(first, cand))

            def superchunk(sj, carry):
                cnt_s = jnp.max(carry[0], axis=0)

                def live(c):
                    for jj in range(16):
                        c = scan_body(sj * 16 + jj, c)
                    return c

                return lax.cond(cnt_s < K, live, lambda c: c, carry)

            cntv, firstv = lax.fori_loop(
                0, N // 256, superchunk,
                (jnp.zeros((16,), jnp.int32), big))
            for h in range(2):
                v = idxbuf[pl.ds(h * 16, 16)]
                pos = iota16 + (h * 16)
                isel[ci][pl.ds(h * 16, 16)] = jnp.where(pos < cntv, v, firstv)
            copies.append(pltpu.async_copy(
                z_hbm.at[isel[ci]], gbuf.at[pl.ds(ci * K, K)], sem))
        for cp in copies:
            cp.wait()
        pltpu.sync_copy(gbuf, out_hbm.at[pl.ds(srow0 * K, CH * K)])
        return _

    lax.fori_loop(0, ROWS_PER_W // CH, chunk, 0)


def _sc_gather(dist2d, z2d):
    mesh = plsc.VectorSubcoreMesh(core_axis_name="c", subcore_axis_name="s")
    fn = functools.partial(
        pl.kernel,
        mesh=mesh,
        compiler_params=pltpu.CompilerParams(needs_layout_passes=False),
        out_type=jax.ShapeDtypeStruct((M, 128), jnp.float32),
        scratch_types=[
            pltpu.VMEM((CH, N), jnp.float32),
            pltpu.VMEM((48,), jnp.int32),
            pltpu.VMEM((K,), jnp.int32),
            pltpu.VMEM((K,), jnp.int32),
            pltpu.VMEM((K,), jnp.int32),
            pltpu.VMEM((K,), jnp.int32),
            pltpu.VMEM((CH * K, 128), jnp.float32),
            pltpu.SemaphoreType.DMA,
        ],
    )(_sc_body)
    return fn(dist2d, z2d)


def _p1_body(g_ref, q_ref, sums_ref):
    i = pl.program_id(0)
    x1 = g_ref[...].reshape(16, K, 128) - q_ref[...].reshape(16, 1, 128)
    x1 = x1.reshape(512, 128)
    s = jnp.sum(x1, axis=0, keepdims=True)
    sq = jnp.sum(x1 * x1, axis=0, keepdims=True)

    @pl.when(i == 0)
    def _():
        sums_ref[...] = jnp.zeros_like(sums_ref)

    sums_ref[0:1, :] += s
    sums_ref[1:2, :] += sq


def _run_p1(g, q):
    return pl.pallas_call(
        _p1_body,
        grid=(M // 512,),
        in_specs=[
            pl.BlockSpec((512, 128), lambda i: (i, 0)),
            pl.BlockSpec((16, 128), lambda i: (i, 0)),
        ],
        out_specs=pl.BlockSpec((8, 128), lambda i: (0, 0)),
        out_shape=jax.ShapeDtypeStruct((8, 128), jnp.float32),
    )(g, q)


def _bn_scale_shift(sums_ref, g_ref, be_ref, m_count):
    mean = sums_ref[0:1, :] * (1.0 / m_count)
    var = sums_ref[1:2, :] * (1.0 / m_count) - mean * mean
    rstd = lax.rsqrt(var + EPS)
    scale = g_ref[...] * rstd
    shift = be_ref[...] - mean * scale
    return scale, shift


def _p2_body(g_ref, q_ref, sums_ref, g1_ref, be1_ref, w2_ref, b2_ref,
             x2_ref, sums2_ref):
    i = pl.program_id(0)
    scale, shift = _bn_scale_shift(sums_ref, g1_ref, be1_ref, float(M))
    x1 = g_ref[...].reshape(16, K, 128) - q_ref[...].reshape(16, 1, 128)
    x1 = x1.reshape(512, 128)
    x1 = jnp.maximum(x1 * scale + shift, 0.0)
    y = lax.dot_general(x1.astype(jnp.bfloat16),
                        w2_ref[...].astype(jnp.bfloat16),
                        (((1,), (1,)), ((), ())),
                        preferred_element_type=jnp.float32) + b2_ref[...]
    x2_ref[...] = y.astype(jnp.bfloat16)
    s = jnp.sum(y, axis=0, keepdims=True)
    sq = jnp.sum(y * y, axis=0, keepdims=True)

    @pl.when(i == 0)
    def _():
        sums2_ref[...] = jnp.zeros_like(sums2_ref)

    sums2_ref[0:1, :] += s
    sums2_ref[1:2, :] += sq


def _run_p2(g, q, sums1, g1, be1, w2, b2):
    return pl.pallas_call(
        _p2_body,
        grid=(M // 512,),
        in_specs=[
            pl.BlockSpec((512, 128), lambda i: (i, 0)),
            pl.BlockSpec((16, 128), lambda i: (i, 0)),
            pl.BlockSpec((8, 128), lambda i: (0, 0)),
            pl.BlockSpec((1, 128), lambda i: (0, 0)),
            pl.BlockSpec((1, 128), lambda i: (0, 0)),
            pl.BlockSpec((128, 128), lambda i: (0, 0)),
            pl.BlockSpec((1, 128), lambda i: (0, 0)),
        ],
        out_specs=[
            pl.BlockSpec((512, 128), lambda i: (i, 0)),
            pl.BlockSpec((8, 128), lambda i: (0, 0)),
        ],
        out_shape=[
            jax.ShapeDtypeStruct((M, 128), jnp.bfloat16),
            jax.ShapeDtypeStruct((8, 128), jnp.float32),
        ],
    )(g, q, sums1, g1, be1, w2, b2)


def _p3_body(x2_ref, sums_ref, g2_ref, be2_ref, w3_ref, b3_ref,
             x3_ref, sums3_ref):
    i = pl.program_id(0)
    scale, shift = _bn_scale_shift(sums_ref, g2_ref, be2_ref, float(M))
    x2 = jnp.maximum(x2_ref[...].astype(jnp.float32) * scale + shift, 0.0)
    y = lax.dot_general(x2.astype(jnp.bfloat16),
                        w3_ref[...].astype(jnp.bfloat16),
                        (((1,), (1,)), ((), ())),
                        preferred_element_type=jnp.float32) + b3_ref[...]
    x3_ref[...] = y.astype(jnp.bfloat16)
    s = jnp.sum(y, axis=0, keepdims=True)
    sq = jnp.sum(y * y, axis=0, keepdims=True)

    @pl.when(i == 0)
    def _():
        sums3_ref[...] = jnp.zeros_like(sums3_ref)

    sums3_ref[0:1, :] += s
    sums3_ref[1:2, :] += sq


def _run_p3(x2, sums2, g2, be2, w3, b3):
    return pl.pallas_call(
        _p3_body,
        grid=(M // 512,),
        in_specs=[
            pl.BlockSpec((512, 128), lambda i: (i, 0)),
            pl.BlockSpec((8, 128), lambda i: (0, 0)),
            pl.BlockSpec((1, 128), lambda i: (0, 0)),
            pl.BlockSpec((1, 128), lambda i: (0, 0)),
            pl.BlockSpec((256, 128), lambda i: (0, 0)),
            pl.BlockSpec((1, 256), lambda i: (0, 0)),
        ],
        out_specs=[
            pl.BlockSpec((512, 256), lambda i: (i, 0)),
            pl.BlockSpec((8, 256), lambda i: (0, 0)),
        ],
        out_shape=[
            jax.ShapeDtypeStruct((M, 256), jnp.bfloat16),
            jax.ShapeDtypeStruct((8, 256), jnp.float32),
        ],
    )(x2, sums2, g2, be2, w3, b3)


def _p4_body(x3_ref, sums_ref, g3_ref, be3_ref, out_ref):
    scale, shift = _bn_scale_shift(sums_ref, g3_ref, be3_ref, float(M))
    x3 = jnp.maximum(x3_ref[...].astype(jnp.float32) * scale + shift, 0.0)
    out_ref[...] = jnp.max(x3.reshape(16, K, 256), axis=1)


def _run_p4(x3, sums3, g3, be3):
    return pl.pallas_call(
        _p4_body,
        grid=(M // 512,),
        in_specs=[
            pl.BlockSpec((512, 256), lambda i: (i, 0)),
            pl.BlockSpec((8, 256), lambda i: (0, 0)),
            pl.BlockSpec((1, 256), lambda i: (0, 0)),
            pl.BlockSpec((1, 256), lambda i: (0, 0)),
        ],
        out_specs=pl.BlockSpec((16, 256), lambda i: (i, 0)),
        out_shape=jax.ShapeDtypeStruct((B * S, 256), jnp.float32),
    )(x3, sums3, g3, be3)


def kernel(xyz, points, W1, b1, g1, be1, W2, b2, g2, be2, W3, b3, g3, be3):
    w1x = W1[:, :3]
    w1p = W1[:, 3:]
    xyzt3 = jnp.transpose(xyz, (0, 2, 1))
    p24 = jnp.transpose(xyzt3, (1, 0, 2)).reshape(24, N)

    nout = _run_fps(p24)
    new_xyz = jnp.transpose(nout.reshape(S, 3, B), (2, 0, 1))
    z = _run_z(xyz, points, w1x, w1p, b1.reshape(1, 128))
    dist, q = _run_dist(new_xyz, xyz, xyzt3, w1x)

    g = _sc_gather(dist.reshape(B * S, N), z.reshape(B * N, 128))

    q2d = q.reshape(B * S, 128)
    sums1 = _run_p1(g, q2d)
    x2, sums2 = _run_p2(g, q2d, sums1, g1.reshape(1, 128), be1.reshape(1, 128),
                        W2, b2.reshape(1, 128))
    x3, sums3 = _run_p3(x2, sums2, g2.reshape(1, 128), be2.reshape(1, 128),
                        W3, b3.reshape(1, 256))
    out = _run_p4(x3, sums3, g3.reshape(1, 256), be3.reshape(1, 256))
    return (new_xyz, out.reshape(B, S, 256))
